# baseline (speedup 1.0000x reference)
import functools

import jax
import jax.numpy as jnp
from jax import lax
from jax.experimental import pallas as pl
from jax.experimental.pallas import tpu as pltpu
from jax.experimental.pallas import tpu_sc as plsc

N = 10000
E = 320000
D = 128
R = 8

CW = 32
NFC = D // CW
NCC = NFC + 1

NSC = 2
NT = 16
RPC = R // NSC
K = 128
NB = 160
EPT = NB * K
EPAD = NT * EPT
SHIFT = 14

ACC_R = RPC * N + 16
TRASH = RPC * N
ZPT = ACC_R // NT
CRT = RPC * N // NT


def _pack_body(dst_ref, rel_ref, src_ref, o_ref):
    dv = dst_ref[...]
    rv = rel_ref[...]
    sv = src_ref[...]
    for c in range(NSC):
        rlv = rv - c * RPC
        valid = (rlv >= 0) & (rlv < RPC)
        row = jnp.where(valid, rlv * N + dv, TRASH)
        o_ref[c] = (row << SHIFT) | sv


def _build_packed(dst_p, rel_p, src_p):
    return pl.pallas_call(
        _pack_body,
        out_shape=jax.ShapeDtypeStruct((NSC, NT, NB, K), jnp.int32),
    )(dst_p, rel_p, src_p)


def _sc_agg_body(xa, xb, xc, xd, packed_hbm, zeros_hbm, agg_hbm,
                 packed2, rows0, rows1, onesb, sb0, sb1, ib0, ib1, acc,
                 gsem0, gsem1):
    c = lax.axis_index("c")
    s = lax.axis_index("s")
    rows = (rows0, rows1)
    sbufs = (sb0, sb1)
    ibufs = (ib0, ib1)
    gsems = (gsem0, gsem1)

    col16 = jnp.where(lax.iota(jnp.int32, 16) == 0,
                      jnp.float32(1), jnp.float32(0))
    zeros16 = jnp.zeros((16,), jnp.float32)

    def ones_body(i, _):
        onesb[i, pl.ds(0, 16)] = col16
        onesb[i, pl.ds(16, 16)] = zeros16
        return 0
    lax.fori_loop(0, K, ones_body, 0)

    pltpu.sync_copy(packed_hbm.at[c, s], packed2)

    def unpack(b, par, need_src):
        for j in range(K // 16):
            pv = packed2[b, pl.ds(j * 16, 16)]
            ibufs[par][pl.ds(j * 16, 16)] = pv >> SHIFT
            if need_src:
                sbufs[par][pl.ds(j * 16, 16)] = pv & ((1 << SHIFT) - 1)

    for cc in range(NFC):
        xin = (xa, xb, xc, xd)[cc]

        pltpu.sync_copy(zeros_hbm, acc.at[pl.ds(s * ZPT, ZPT)])
        plsc.subcore_barrier()

        unpack(0, 0, True)
        pltpu.async_copy(xin.at[sb0], rows0, gsem0)

        def bb_body(bb, _):
            for par in range(2):
                b = bb * 2 + par
                rbuf, obuf = rows[par], rows[1 - par]
                pltpu.make_async_copy(xin.at[sbufs[par]], rbuf,
                                      gsems[par]).wait()
                unpack(b + 1, 1 - par, True)
                pltpu.async_copy(xin.at[sbufs[1 - par]], obuf,
                                 gsems[1 - par])
                pltpu.sync_copy(rbuf, acc.at[ibufs[par]], add=True)
            return 0
        lax.fori_loop(0, (NB - 2) // 2, bb_body, 0)
        pltpu.make_async_copy(xin.at[sb0], rows0, gsem0).wait()
        unpack(NB - 1, 1, True)
        pltpu.async_copy(xin.at[sb1], rows1, gsem1)
        pltpu.sync_copy(rows0, acc.at[ib0], add=True)
        pltpu.make_async_copy(xin.at[sb1], rows1, gsem1).wait()
        pltpu.sync_copy(rows1, acc.at[ib1], add=True)
        plsc.subcore_barrier()

        rloc = s // 4
        n0 = (s % 4) * CRT
        pltpu.sync_copy(acc.at[pl.ds(s * CRT, CRT)],
                        agg_hbm.at[cc, c * RPC + rloc, pl.ds(n0, CRT)])
        plsc.subcore_barrier()

    pltpu.sync_copy(zeros_hbm, acc.at[pl.ds(s * ZPT, ZPT)])
    plsc.subcore_barrier()

    def cb_body(b, _):
        unpack(b, 0, False)
        pltpu.sync_copy(onesb, acc.at[ib0], add=True)
        return 0
    lax.fori_loop(0, NB, cb_body, 0)
    plsc.subcore_barrier()
    rloc = s // 4
    n0 = (s % 4) * CRT
    pltpu.sync_copy(acc.at[pl.ds(s * CRT, CRT)],
                    agg_hbm.at[NFC, c * RPC + rloc, pl.ds(n0, CRT)])
    plsc.subcore_barrier()


_sc_agg = pl.kernel(
    _sc_agg_body,
    out_type=jax.ShapeDtypeStruct((NCC, R, N, CW), jnp.float32),
    mesh=plsc.VectorSubcoreMesh(
        core_axis_name="c", subcore_axis_name="s",
        num_cores=NSC, num_subcores=NT),
    scratch_types=[
        pltpu.VMEM((NB, K), jnp.int32),
        pltpu.VMEM((K, CW), jnp.float32),
        pltpu.VMEM((K, CW), jnp.float32),
        pltpu.VMEM((K, CW), jnp.float32),
        pltpu.VMEM((K,), jnp.int32),
        pltpu.VMEM((K,), jnp.int32),
        pltpu.VMEM((K,), jnp.int32),
        pltpu.VMEM((K,), jnp.int32),
        pltpu.VMEM_SHARED((ACC_R, CW), jnp.float32),
        pltpu.SemaphoreType.DMA,
        pltpu.SemaphoreType.DMA,
    ],
    compiler_params=pltpu.CompilerParams(use_tc_tiling_on_sc=False),
)


def _tc_layer_body(apply_gelu, xa_ref, xb_ref, xc_ref, xd_ref, agg_ref,
                   w_ref, root_ref, bias_ref, *out_refs):
    xfull = jnp.concatenate(
        [xa_ref[...], xb_ref[...], xc_ref[...], xd_ref[...]], axis=1)
    acc = jnp.dot(xfull, root_ref[...], preferred_element_type=jnp.float32)
    for r in range(R):
        a = jnp.concatenate(
            [agg_ref[0, r], agg_ref[1, r], agg_ref[2, r], agg_ref[3, r]],
            axis=1)
        scale = 1.0 / jnp.maximum(agg_ref[NFC, r][:, 0:1], 1.0)
        acc += jnp.dot(a * scale, w_ref[r],
                       preferred_element_type=jnp.float32)
    acc = acc + bias_ref[...]
    if apply_gelu:
        acc = acc * 0.5 * (1.0 + lax.erf(acc * (2.0 ** -0.5)))
        for q in range(NFC):
            out_refs[q][...] = acc[:, q * CW:(q + 1) * CW]
    else:
        out_refs[0][...] = acc


def _tc_layer(xs, agg, weight, root, bias, apply_gelu):
    BN = 400
    if apply_gelu:
        out_shape = [jax.ShapeDtypeStruct((N, CW), jnp.float32)] * NFC
        out_specs = [pl.BlockSpec((BN, CW), lambda i: (i, 0))] * NFC
    else:
        out_shape = jax.ShapeDtypeStruct((N, D), jnp.float32)
        out_specs = pl.BlockSpec((BN, D), lambda i: (i, 0))
    return pl.pallas_call(
        functools.partial(_tc_layer_body, apply_gelu),
        grid=(N // BN,),
        in_specs=[
            pl.BlockSpec((BN, CW), lambda i: (i, 0)),
            pl.BlockSpec((BN, CW), lambda i: (i, 0)),
            pl.BlockSpec((BN, CW), lambda i: (i, 0)),
            pl.BlockSpec((BN, CW), lambda i: (i, 0)),
            pl.BlockSpec((NCC, R, BN, CW), lambda i: (0, 0, i, 0)),
            pl.BlockSpec((R, D, D), lambda i: (0, 0, 0)),
            pl.BlockSpec((D, D), lambda i: (0, 0)),
            pl.BlockSpec((1, D), lambda i: (0, 0)),
        ],
        out_specs=out_specs,
        out_shape=out_shape,
    )(*xs, agg, weight, root, bias)


def kernel(embs, edge_index, rel_type, batch_size, weight1, root1, bias1,
           weight2, root2, bias2):
    src = edge_index[0]
    dst = edge_index[1]
    pad = EPAD - E
    src_p = jnp.concatenate(
        [src, jnp.zeros((pad,), jnp.int32)]).reshape(NT, NB, K)
    dst_p = jnp.concatenate(
        [dst, jnp.zeros((pad,), jnp.int32)]).reshape(NT, NB, K)
    rel_p = jnp.concatenate(
        [rel_type, jnp.full((pad,), R, jnp.int32)]).reshape(NT, NB, K)
    packed = _build_packed(dst_p, rel_p, src_p)
    zeros_acc = jnp.zeros((ZPT, CW), jnp.float32)

    xs = tuple(embs[:, q * CW:(q + 1) * CW] for q in range(NFC))

    agg1 = _sc_agg(*xs, packed, zeros_acc)
    xs1 = _tc_layer(xs, agg1, weight1, root1, bias1.reshape(1, D), True)
    agg2 = _sc_agg(*xs1, packed, zeros_acc)
    out = _tc_layer(xs1, agg2, weight2, root2, bias2.reshape(1, D), False)
    return out


# speedup vs baseline: 1.7800x; 1.7800x over previous
import functools

import jax
import jax.numpy as jnp
from jax import lax
from jax.experimental import pallas as pl
from jax.experimental.pallas import tpu as pltpu
from jax.experimental.pallas import tpu_sc as plsc

N = 10000
E = 320000
D = 128
R = 8

CW = 32
NFC = D // CW
NCC = NFC + 1
BF = jnp.bfloat16

NSC = 2
NT = 16
K = 128
NB = 80
EPT = NB * K
EPAD = NSC * NT * EPT
SHIFT = 14

ACC_R = R * N + 16
TRASH = R * N
ZPT = ACC_R // NT
CRT = R * N // NT


def _pack_body(dst_ref, rel_ref, src_ref, o_ref):
    dv = dst_ref[...]
    rv = rel_ref[...]
    sv = src_ref[...]
    row = jnp.where(rv < R, rv * N + dv, TRASH)
    o_ref[...] = (row << SHIFT) | sv


def _build_packed(dst_p, rel_p, src_p):
    return pl.pallas_call(
        _pack_body,
        out_shape=jax.ShapeDtypeStruct((NSC, NT, NB, K), jnp.int32),
    )(dst_p, rel_p, src_p)


def _sc_agg_body(xa, xb, xc, xd, packed_hbm, zeros_hbm, ones_hbm, agg_hbm,
                 packed2, rows0, rows1, onesb, sb0, sb1, ib0, ib1, acc,
                 gsem0, gsem1):
    c = lax.axis_index("c")
    s = lax.axis_index("s")
    rows = (rows0, rows1)
    sbufs = (sb0, sb1)
    ibufs = (ib0, ib1)
    gsems = (gsem0, gsem1)

    pltpu.sync_copy(ones_hbm, onesb)
    pltpu.sync_copy(packed_hbm.at[c, s], packed2)

    def unpack(b, par, need_src):
        for j in range(K // 16):
            pv = packed2[b, pl.ds(j * 16, 16)]
            ibufs[par][pl.ds(j * 16, 16)] = pv >> SHIFT
            if need_src:
                sbufs[par][pl.ds(j * 16, 16)] = pv & ((1 << SHIFT) - 1)

    def copyout(cc):
        rr = s // 2
        n0 = (s % 2) * CRT
        pltpu.sync_copy(acc.at[pl.ds(s * CRT, CRT)],
                        agg_hbm.at[cc, c, rr, pl.ds(n0, CRT)])

    for cc in range(NFC):
        xin = (xa, xb, xc, xd)[cc]

        pltpu.sync_copy(zeros_hbm, acc.at[pl.ds(s * ZPT, ZPT)])
        plsc.subcore_barrier()

        unpack(0, 0, True)
        pltpu.async_copy(xin.at[sb0], rows0, gsem0)

        def bb_body(bb, _):
            for par in range(2):
                b = bb * 2 + par
                rbuf, obuf = rows[par], rows[1 - par]
                pltpu.make_async_copy(xin.at[sbufs[par]], rbuf,
                                      gsems[par]).wait()
                unpack(b + 1, 1 - par, True)
                pltpu.async_copy(xin.at[sbufs[1 - par]], obuf,
                                 gsems[1 - par])
                pltpu.sync_copy(rbuf, acc.at[ibufs[par]], add=True)
            return 0
        lax.fori_loop(0, (NB - 2) // 2, bb_body, 0)
        pltpu.make_async_copy(xin.at[sb0], rows0, gsem0).wait()
        unpack(NB - 1, 1, True)
        pltpu.async_copy(xin.at[sb1], rows1, gsem1)
        pltpu.sync_copy(rows0, acc.at[ib0], add=True)
        pltpu.make_async_copy(xin.at[sb1], rows1, gsem1).wait()
        pltpu.sync_copy(rows1, acc.at[ib1], add=True)
        plsc.subcore_barrier()

        copyout(cc)
        plsc.subcore_barrier()

    pltpu.sync_copy(zeros_hbm, acc.at[pl.ds(s * ZPT, ZPT)])
    plsc.subcore_barrier()

    def cb_body(b, _):
        unpack(b, 0, False)
        pltpu.sync_copy(onesb, acc.at[ib0], add=True)
        return 0
    lax.fori_loop(0, NB, cb_body, 0)
    plsc.subcore_barrier()
    copyout(NFC)
    plsc.subcore_barrier()


_sc_agg = pl.kernel(
    _sc_agg_body,
    out_type=jax.ShapeDtypeStruct((NCC, NSC, R, N, CW), BF),
    mesh=plsc.VectorSubcoreMesh(
        core_axis_name="c", subcore_axis_name="s",
        num_cores=NSC, num_subcores=NT),
    scratch_types=[
        pltpu.VMEM((NB, K), jnp.int32),
        pltpu.VMEM((K, CW), BF),
        pltpu.VMEM((K, CW), BF),
        pltpu.VMEM((K, CW), BF),
        pltpu.VMEM((K,), jnp.int32),
        pltpu.VMEM((K,), jnp.int32),
        pltpu.VMEM((K,), jnp.int32),
        pltpu.VMEM((K,), jnp.int32),
        pltpu.VMEM_SHARED((ACC_R, CW), BF),
        pltpu.SemaphoreType.DMA,
        pltpu.SemaphoreType.DMA,
    ],
    compiler_params=pltpu.CompilerParams(use_tc_tiling_on_sc=False),
)


def _tc_layer_body(apply_gelu, xa_ref, xb_ref, xc_ref, xd_ref, agg_ref,
                   w_ref, root_ref, bias_ref, *out_refs):
    xfull = jnp.concatenate(
        [xa_ref[...], xb_ref[...], xc_ref[...], xd_ref[...]],
        axis=1).astype(jnp.float32)
    acc = jnp.dot(xfull, root_ref[...], preferred_element_type=jnp.float32)
    for r in range(R):
        a = jnp.concatenate(
            [agg_ref[q, 0, r].astype(jnp.float32)
             + agg_ref[q, 1, r].astype(jnp.float32) for q in range(NFC)],
            axis=1)
        cnt = (agg_ref[NFC, 0, r][:, 0:1].astype(jnp.float32)
               + agg_ref[NFC, 1, r][:, 0:1].astype(jnp.float32))
        scale = 1.0 / jnp.maximum(cnt, 1.0)
        acc += jnp.dot(a * scale, w_ref[r],
                       preferred_element_type=jnp.float32)
    acc = acc + bias_ref[...]
    if apply_gelu:
        acc = acc * 0.5 * (1.0 + lax.erf(acc * (2.0 ** -0.5)))
        for q in range(NFC):
            out_refs[q][...] = acc[:, q * CW:(q + 1) * CW].astype(BF)
    else:
        out_refs[0][...] = acc


def _tc_layer(xs, agg, weight, root, bias, apply_gelu):
    BN = 400
    if apply_gelu:
        out_shape = [jax.ShapeDtypeStruct((N, CW), BF)] * NFC
        out_specs = [pl.BlockSpec((BN, CW), lambda i: (i, 0))] * NFC
    else:
        out_shape = jax.ShapeDtypeStruct((N, D), jnp.float32)
        out_specs = pl.BlockSpec((BN, D), lambda i: (i, 0))
    return pl.pallas_call(
        functools.partial(_tc_layer_body, apply_gelu),
        grid=(N // BN,),
        in_specs=[
            pl.BlockSpec((BN, CW), lambda i: (i, 0)),
            pl.BlockSpec((BN, CW), lambda i: (i, 0)),
            pl.BlockSpec((BN, CW), lambda i: (i, 0)),
            pl.BlockSpec((BN, CW), lambda i: (i, 0)),
            pl.BlockSpec((NCC, NSC, R, BN, CW), lambda i: (0, 0, 0, i, 0)),
            pl.BlockSpec((R, D, D), lambda i: (0, 0, 0)),
            pl.BlockSpec((D, D), lambda i: (0, 0)),
            pl.BlockSpec((1, D), lambda i: (0, 0)),
        ],
        out_specs=out_specs,
        out_shape=out_shape,
    )(*xs, agg, weight, root, bias)


def kernel(embs, edge_index, rel_type, batch_size, weight1, root1, bias1,
           weight2, root2, bias2):
    src = edge_index[0]
    dst = edge_index[1]
    pad = EPAD - E
    src_p = jnp.concatenate(
        [src, jnp.zeros((pad,), jnp.int32)]).reshape(NSC, NT, NB, K)
    dst_p = jnp.concatenate(
        [dst, jnp.zeros((pad,), jnp.int32)]).reshape(NSC, NT, NB, K)
    rel_p = jnp.concatenate(
        [rel_type, jnp.full((pad,), R, jnp.int32)]).reshape(NSC, NT, NB, K)
    packed = _build_packed(dst_p, rel_p, src_p)
    zeros_acc = jnp.zeros((ZPT, CW), BF)
    ones_col = jnp.zeros((K, CW), BF).at[:, 0].set(1)

    xs = tuple(embs[:, q * CW:(q + 1) * CW].astype(BF) for q in range(NFC))

    agg1 = _sc_agg(*xs, packed, zeros_acc, ones_col)
    xs1 = _tc_layer(xs, agg1, weight1, root1, bias1.reshape(1, D), True)
    agg2 = _sc_agg(*xs1, packed, zeros_acc, ones_col)
    out = _tc_layer(xs1, agg2, weight2, root2, bias2.reshape(1, D), False)
    return out
